# baseline (speedup 1.0000x reference)
import functools

import jax
import jax.numpy as jnp
from jax import lax
from jax.experimental import pallas as pl
from jax.experimental.pallas import tpu as pltpu
from jax.experimental.pallas import tpu_sc as plsc

N = 10000
DIN = 128
DH = 128
DOUT = 10
G = 64
E = 320000

NC = 2
NS = 16
NW = NC * NS
C = 80
NCH = 126
TH = 42
EPT = NCH * C
EPAD = NW * EPT
RPT = 624


def _deg_body(dst_hbm, w_hbm, deg0_hbm, deg1_hbm, dst_v, w_v, zero_v, acc_sh,
              dsem):
    cid = lax.axis_index("c")
    sid = lax.axis_index("s")
    wid = cid * NS + sid

    pltpu.sync_copy(dst_hbm.at[wid], dst_v)
    pltpu.sync_copy(w_hbm.at[wid], w_v)

    zf = jnp.zeros((16,), jnp.float32)

    def zfill(i, carry):
        zero_v[pl.ds(pl.multiple_of(i * 16, 16), 16)] = zf
        return carry

    lax.fori_loop(0, 40, zfill, 0)

    @pl.when(sid == 0)
    def _():
        for i in range(15):
            pltpu.sync_copy(zero_v, acc_sh.at[pl.ds(i * 640, 640)])
        pltpu.sync_copy(zero_v.at[pl.ds(0, 400)], acc_sh.at[pl.ds(9600, 400)])

    plsc.subcore_barrier()

    for h in range(3):
        for q in range(3):
            def issue(jj, carry, h=h, q=q):
                j = q * (TH // 3) + jj
                pltpu.async_copy(w_v.at[h, j], acc_sh.at[dst_v.at[h, j]],
                                 dsem, add=True)
                return carry

            lax.fori_loop(0, TH // 3, issue, 0)

            def drain(jj, carry, h=h, q=q):
                j = q * (TH // 3) + jj
                pltpu.make_async_copy(w_v.at[h, j],
                                      acc_sh.at[dst_v.at[h, j]],
                                      dsem).wait()
                return carry

            lax.fori_loop(0, TH // 3, drain, 0)

    plsc.subcore_barrier()

    @pl.when(jnp.logical_and(sid == 0, cid == 0))
    def _():
        pltpu.sync_copy(acc_sh, deg0_hbm)

    @pl.when(jnp.logical_and(sid == 0, cid == 1))
    def _():
        pltpu.sync_copy(acc_sh, deg1_hbm)


_deg_call = functools.partial(
    pl.kernel,
    out_type=(
        jax.ShapeDtypeStruct((N,), jnp.float32),
        jax.ShapeDtypeStruct((N,), jnp.float32),
    ),
    mesh=plsc.VectorSubcoreMesh(core_axis_name="c", subcore_axis_name="s"),
    scratch_types=[
        pltpu.VMEM((3, TH, C), jnp.int32),
        pltpu.VMEM((3, TH, C), jnp.float32),
        pltpu.VMEM((640,), jnp.float32),
        pltpu.VMEM_SHARED((N,), jnp.float32),
        pltpu.SemaphoreType.DMA,
    ],
)(_deg_body)


def _agg_body(y_hbm, src_hbm, dst_hbm, w_hbm, out0_hbm, out1_hbm,
              src_v, dst_v, w_v, rows_v, acc_sh,
              gsem0, gsem1, gsem2, ssem0, ssem1, ssem2):
    cid = lax.axis_index("c")
    sid = lax.axis_index("s")
    wid = cid * NS + sid
    gsems = (gsem0, gsem1, gsem2)
    ssems = (ssem0, ssem1, ssem2)

    zf = jnp.zeros((16,), jnp.float32)

    def zfill(i, carry):
        for bb in range(3):
            for f in range(8):
                rows_v[bb, i, pl.ds(f * 16, 16)] = zf
        return carry

    lax.fori_loop(0, C, zfill, 0)

    base = pl.multiple_of(sid * RPT, 8)
    for k in range(7):
        pltpu.async_copy(rows_v.at[0], acc_sh.at[pl.ds(base + k * 80, 80)],
                         gsem0)

    @pl.when(sid < 15)
    def _():
        pltpu.async_copy(rows_v.at[0, pl.ds(0, 64)],
                         acc_sh.at[pl.ds(base + 560, 64)], gsem1)

    @pl.when(sid == 15)
    def _():
        pltpu.async_copy(rows_v.at[0], acc_sh.at[pl.ds(base + 560, 80)],
                         gsem1)

    for k in range(7):
        pltpu.make_async_copy(rows_v.at[0],
                              acc_sh.at[pl.ds(base + k * 80, 80)],
                              gsem0).wait()

    @pl.when(sid < 15)
    def _():
        pltpu.make_async_copy(rows_v.at[0, pl.ds(0, 64)],
                              acc_sh.at[pl.ds(base + 560, 64)], gsem1).wait()

    @pl.when(sid == 15)
    def _():
        pltpu.make_async_copy(rows_v.at[0],
                              acc_sh.at[pl.ds(base + 560, 80)], gsem1).wait()

    plsc.subcore_barrier()

    def step(t, carry):
        jj = t * 3
        for b3 in range(3):
            j = jj + b3

            pltpu.make_async_copy(y_hbm.at[src_v.at[j]], rows_v.at[b3],
                                  gsems[b3]).wait()

            for g in range(C // 16):
                wv = w_v[j, pl.ds(g * 16, 16)]
                for lane in range(16):
                    sv = jnp.full((16,), wv[lane], jnp.float32)
                    e = g * 16 + lane
                    for f in range(8):
                        rows_v[b3, e, pl.ds(f * 16, 16)] = (
                            rows_v[b3, e, pl.ds(f * 16, 16)] * sv)

            @pl.when(j >= 1)
            def _():
                pltpu.make_async_copy(
                    rows_v.at[(b3 + 2) % 3], acc_sh.at[dst_v.at[j - 1]],
                    ssems[(b3 + 2) % 3]).wait()

            @pl.when(j + 2 < TH)
            def _():
                pltpu.async_copy(y_hbm.at[src_v.at[j + 2]],
                                 rows_v.at[(b3 + 2) % 3], gsems[(b3 + 2) % 3])

            pltpu.async_copy(rows_v.at[b3], acc_sh.at[dst_v.at[j]],
                             ssems[b3], add=True)
        return carry

    for h in range(3):
        pltpu.sync_copy(src_hbm.at[wid, h], src_v)
        pltpu.sync_copy(dst_hbm.at[wid, h], dst_v)
        pltpu.sync_copy(w_hbm.at[wid, h], w_v)
        pltpu.async_copy(y_hbm.at[src_v.at[0]], rows_v.at[0], gsems[0])
        pltpu.async_copy(y_hbm.at[src_v.at[1]], rows_v.at[1], gsems[1])
        lax.fori_loop(0, TH // 3, step, 0)
        pltpu.make_async_copy(rows_v.at[(TH - 1) % 3],
                              acc_sh.at[dst_v.at[TH - 1]],
                              ssems[(TH - 1) % 3]).wait()

    plsc.subcore_barrier()

    row0 = pl.multiple_of(sid * RPT, 8)
    for out_hbm, this_cid in ((out0_hbm, 0), (out1_hbm, 1)):
        @pl.when(jnp.logical_and(cid == this_cid, sid < 15))
        def _():
            pltpu.sync_copy(acc_sh.at[pl.ds(row0, RPT)],
                            out_hbm.at[pl.ds(row0, RPT)])

        @pl.when(jnp.logical_and(cid == this_cid, sid == 15))
        def _():
            pltpu.sync_copy(acc_sh.at[pl.ds(row0, 640)],
                            out_hbm.at[pl.ds(row0, 640)])


_agg_call = functools.partial(
    pl.kernel,
    out_type=(
        jax.ShapeDtypeStruct((N, DH), jnp.float32),
        jax.ShapeDtypeStruct((N, DH), jnp.float32),
    ),
    mesh=plsc.VectorSubcoreMesh(core_axis_name="c", subcore_axis_name="s"),
    scratch_types=[
        pltpu.VMEM((TH, C), jnp.int32),
        pltpu.VMEM((TH, C), jnp.int32),
        pltpu.VMEM((TH, C), jnp.float32),
        pltpu.VMEM((3, C, DH), jnp.float32),
        pltpu.VMEM_SHARED((N, DH), jnp.float32),
        pltpu.SemaphoreType.DMA,
        pltpu.SemaphoreType.DMA,
        pltpu.SemaphoreType.DMA,
        pltpu.SemaphoreType.DMA,
        pltpu.SemaphoreType.DMA,
        pltpu.SemaphoreType.DMA,
    ],
)(_agg_body)


def _tca_body(x_ref, w1_ref, d0_ref, d1_ref, y1_ref, dinv_ref):
    deg = d0_ref[...] + d1_ref[...] + 1.0
    dinv = lax.rsqrt(deg)
    xw = jnp.dot(x_ref[...], w1_ref[...], preferred_element_type=jnp.float32)
    y1_ref[...] = xw * dinv
    dinv_ref[...] = dinv


def _tca(x, W1, d0, d1):
    return pl.pallas_call(
        _tca_body,
        out_shape=(
            jax.ShapeDtypeStruct((N, DH), jnp.float32),
            jax.ShapeDtypeStruct((N, 1), jnp.float32),
        ),
    )(x, W1, d0, d1)


def _tcb_body(a0_ref, a1_ref, y1_ref, dinv_ref, b1_ref, w2_ref, y2_ref):
    dinv = dinv_ref[...]
    h1 = dinv * (a0_ref[...] + a1_ref[...] + y1_ref[...]) + b1_ref[...]
    h1 = jnp.maximum(h1, 0.0)
    y2_ref[...] = jnp.dot(h1, w2_ref[...],
                          preferred_element_type=jnp.float32) * dinv


def _tcb(a0, a1, y1, dinv, b1, W2):
    return pl.pallas_call(
        _tcb_body,
        out_shape=jax.ShapeDtypeStruct((N, DH), jnp.float32),
    )(a0, a1, y1, dinv, b1, W2)


def _tcc_body(a0_ref, a1_ref, y2_ref, dinv_ref, b2_ref, wl_ref, bl_ref,
              wh_ref, bh_ref, bt_ref, out_ref):
    h2 = dinv_ref[...] * (a0_ref[...] + a1_ref[...] + y2_ref[...]) + b2_ref[...]
    hl = jnp.dot(h2, wl_ref[...], preferred_element_type=jnp.float32) + bl_ref[...]
    gid = lax.broadcasted_iota(jnp.int32, (G, N), 0)
    mask = (gid == bt_ref[...]).astype(jnp.float32)
    pooled = jnp.dot(mask, hl, preferred_element_type=jnp.float32)
    cnt = jnp.sum(mask, axis=1, keepdims=True)
    pooled = pooled / jnp.maximum(cnt, 1.0)
    logits = jnp.dot(pooled, wh_ref[...],
                     preferred_element_type=jnp.float32) + bh_ref[...]
    m = jnp.max(logits, axis=1, keepdims=True)
    ex = jnp.exp(logits - m)
    out_ref[...] = ex / jnp.sum(ex, axis=1, keepdims=True)


def _tcc(a0, a1, y2, dinv, b2, Wl, bl, Wh, bh, bt):
    return pl.pallas_call(
        _tcc_body,
        out_shape=jax.ShapeDtypeStruct((G, DOUT), jnp.float32),
    )(a0, a1, y2, dinv, b2, Wl, bl, Wh, bh, bt)


def kernel(x, edge_index, edge_weight, batch_idx, W1, b1, W2, b2, Wl, bl,
           Wh, bh):
    src = edge_index[0]
    dst = edge_index[1]
    npad = EPAD - E
    pad = jnp.arange(npad, dtype=jnp.int32) % N
    srcp = jnp.concatenate([src, pad]).reshape(NW, 3, TH, C)
    dstp = jnp.concatenate([dst, pad]).reshape(NW, 3, TH, C)
    wp = jnp.concatenate(
        [edge_weight, jnp.zeros((npad,), jnp.float32)]).reshape(NW, 3, TH, C)

    d0, d1 = _deg_call(dstp, wp)
    y1, dinv = _tca(x, W1, d0.reshape(N, 1), d1.reshape(N, 1))
    a10, a11 = _agg_call(y1, srcp, dstp, wp)
    y2 = _tcb(a10, a11, y1, dinv, b1.reshape(1, DH), W2)
    a20, a21 = _agg_call(y2, srcp, dstp, wp)
    out = _tcc(a20, a21, y2, dinv, b2.reshape(1, DH), Wl, bl.reshape(1, DH // 2),
               Wh, bh.reshape(1, DOUT), batch_idx.reshape(1, N))
    return out


# speedup vs baseline: 1.3714x; 1.3714x over previous
import functools

import jax
import jax.numpy as jnp
from jax import lax
from jax.experimental import pallas as pl
from jax.experimental.pallas import tpu as pltpu
from jax.experimental.pallas import tpu_sc as plsc

N = 10000
DIN = 128
DH = 128
DOUT = 10
G = 64
E = 320000

NC = 2
NS = 16
NW = NC * NS
C = 80
NCH = 126
TH = 42
EPT = NCH * C
EPAD = NW * EPT
RPT = 624


def _deg_body(dst_hbm, w_hbm, deg0_hbm, deg1_hbm, dst_v, w_v, zero_v, acc_sh,
              dsem):
    cid = lax.axis_index("c")
    sid = lax.axis_index("s")
    wid = cid * NS + sid

    pltpu.sync_copy(dst_hbm.at[wid], dst_v)
    pltpu.sync_copy(w_hbm.at[wid], w_v)

    zf = jnp.zeros((16,), jnp.float32)

    def zfill(i, carry):
        zero_v[pl.ds(pl.multiple_of(i * 16, 16), 16)] = zf
        return carry

    lax.fori_loop(0, 40, zfill, 0)

    @pl.when(sid == 0)
    def _():
        for i in range(15):
            pltpu.sync_copy(zero_v, acc_sh.at[pl.ds(i * 640, 640)])
        pltpu.sync_copy(zero_v.at[pl.ds(0, 400)], acc_sh.at[pl.ds(9600, 400)])

    plsc.subcore_barrier()

    for h in range(3):
        for q in range(3):
            def issue(jj, carry, h=h, q=q):
                j = q * (TH // 3) + jj
                pltpu.async_copy(w_v.at[h, j], acc_sh.at[dst_v.at[h, j]],
                                 dsem, add=True)
                return carry

            lax.fori_loop(0, TH // 3, issue, 0)

            def drain(jj, carry, h=h, q=q):
                j = q * (TH // 3) + jj
                pltpu.make_async_copy(w_v.at[h, j],
                                      acc_sh.at[dst_v.at[h, j]],
                                      dsem).wait()
                return carry

            lax.fori_loop(0, TH // 3, drain, 0)

    plsc.subcore_barrier()

    @pl.when(jnp.logical_and(sid == 0, cid == 0))
    def _():
        pltpu.sync_copy(acc_sh, deg0_hbm)

    @pl.when(jnp.logical_and(sid == 0, cid == 1))
    def _():
        pltpu.sync_copy(acc_sh, deg1_hbm)


_deg_call = functools.partial(
    pl.kernel,
    out_type=(
        jax.ShapeDtypeStruct((N,), jnp.float32),
        jax.ShapeDtypeStruct((N,), jnp.float32),
    ),
    mesh=plsc.VectorSubcoreMesh(core_axis_name="c", subcore_axis_name="s"),
    scratch_types=[
        pltpu.VMEM((3, TH, C), jnp.int32),
        pltpu.VMEM((3, TH, C), jnp.float32),
        pltpu.VMEM((640,), jnp.float32),
        pltpu.VMEM_SHARED((N,), jnp.float32),
        pltpu.SemaphoreType.DMA,
    ],
)(_deg_body)


def _agg_body(y_hbm, src_hbm, dst_hbm, w_hbm, out0_hbm, out1_hbm,
              src_v, dst_v, w_v, rows_v, acc_sh,
              gsem0, gsem1, gsem2, ssem0, ssem1, ssem2):
    cid = lax.axis_index("c")
    sid = lax.axis_index("s")
    wid = cid * NS + sid
    gsems = (gsem0, gsem1, gsem2)
    ssems = (ssem0, ssem1, ssem2)

    zf = jnp.zeros((16,), jnp.float32)

    def zfill(i, carry):
        for bb in range(3):
            for f in range(8):
                rows_v[bb, i, pl.ds(f * 16, 16)] = zf
        return carry

    lax.fori_loop(0, C, zfill, 0)

    base = pl.multiple_of(sid * RPT, 8)
    for k in range(7):
        pltpu.async_copy(rows_v.at[0], acc_sh.at[pl.ds(base + k * 80, 80)],
                         gsem0)

    @pl.when(sid < 15)
    def _():
        pltpu.async_copy(rows_v.at[0, pl.ds(0, 64)],
                         acc_sh.at[pl.ds(base + 560, 64)], gsem1)

    @pl.when(sid == 15)
    def _():
        pltpu.async_copy(rows_v.at[0], acc_sh.at[pl.ds(base + 560, 80)],
                         gsem1)

    for k in range(7):
        pltpu.make_async_copy(rows_v.at[0],
                              acc_sh.at[pl.ds(base + k * 80, 80)],
                              gsem0).wait()

    @pl.when(sid < 15)
    def _():
        pltpu.make_async_copy(rows_v.at[0, pl.ds(0, 64)],
                              acc_sh.at[pl.ds(base + 560, 64)], gsem1).wait()

    @pl.when(sid == 15)
    def _():
        pltpu.make_async_copy(rows_v.at[0],
                              acc_sh.at[pl.ds(base + 560, 80)], gsem1).wait()

    plsc.subcore_barrier()

    def step(t, carry):
        jj = t * 3
        for b3 in range(3):
            j = jj + b3

            pltpu.make_async_copy(y_hbm.at[src_v.at[j]], rows_v.at[b3],
                                  gsems[b3]).wait()

            def scale(g, c2):
                wv = w_v[j, pl.ds(pl.multiple_of(g * 16, 16), 16)]
                for lane in range(16):
                    sv = jnp.full((16,), wv[lane], jnp.float32)
                    e = g * 16 + lane
                    for f in range(8):
                        rows_v[b3, e, pl.ds(f * 16, 16)] = (
                            rows_v[b3, e, pl.ds(f * 16, 16)] * sv)
                return c2

            lax.fori_loop(0, C // 16, scale, 0)

            @pl.when(j >= 1)
            def _():
                pltpu.make_async_copy(
                    rows_v.at[(b3 + 2) % 3], acc_sh.at[dst_v.at[j - 1]],
                    ssems[(b3 + 2) % 3]).wait()

            @pl.when(j + 2 < TH)
            def _():
                pltpu.async_copy(y_hbm.at[src_v.at[j + 2]],
                                 rows_v.at[(b3 + 2) % 3], gsems[(b3 + 2) % 3])

            pltpu.async_copy(rows_v.at[b3], acc_sh.at[dst_v.at[j]],
                             ssems[b3], add=True)
        return carry

    for h in range(3):
        pltpu.sync_copy(src_hbm.at[wid, h], src_v)
        pltpu.sync_copy(dst_hbm.at[wid, h], dst_v)
        pltpu.sync_copy(w_hbm.at[wid, h], w_v)
        pltpu.async_copy(y_hbm.at[src_v.at[0]], rows_v.at[0], gsems[0])
        pltpu.async_copy(y_hbm.at[src_v.at[1]], rows_v.at[1], gsems[1])
        lax.fori_loop(0, TH // 3, step, 0)
        pltpu.make_async_copy(rows_v.at[(TH - 1) % 3],
                              acc_sh.at[dst_v.at[TH - 1]],
                              ssems[(TH - 1) % 3]).wait()

    plsc.subcore_barrier()

    row0 = pl.multiple_of(sid * RPT, 8)
    for out_hbm, this_cid in ((out0_hbm, 0), (out1_hbm, 1)):
        @pl.when(jnp.logical_and(cid == this_cid, sid < 15))
        def _():
            pltpu.sync_copy(acc_sh.at[pl.ds(row0, RPT)],
                            out_hbm.at[pl.ds(row0, RPT)])

        @pl.when(jnp.logical_and(cid == this_cid, sid == 15))
        def _():
            pltpu.sync_copy(acc_sh.at[pl.ds(row0, 640)],
                            out_hbm.at[pl.ds(row0, 640)])


_agg_call = functools.partial(
    pl.kernel,
    out_type=(
        jax.ShapeDtypeStruct((N, DH), jnp.float32),
        jax.ShapeDtypeStruct((N, DH), jnp.float32),
    ),
    mesh=plsc.VectorSubcoreMesh(core_axis_name="c", subcore_axis_name="s"),
    scratch_types=[
        pltpu.VMEM((TH, C), jnp.int32),
        pltpu.VMEM((TH, C), jnp.int32),
        pltpu.VMEM((TH, C), jnp.float32),
        pltpu.VMEM((3, C, DH), jnp.float32),
        pltpu.VMEM_SHARED((N, DH), jnp.float32),
        pltpu.SemaphoreType.DMA,
        pltpu.SemaphoreType.DMA,
        pltpu.SemaphoreType.DMA,
        pltpu.SemaphoreType.DMA,
        pltpu.SemaphoreType.DMA,
        pltpu.SemaphoreType.DMA,
    ],
)(_agg_body)


def _tca_body(x_ref, w1_ref, d0_ref, d1_ref, y1_ref, dinv_ref):
    deg = d0_ref[...] + d1_ref[...] + 1.0
    dinv = lax.rsqrt(deg)
    xw = jnp.dot(x_ref[...], w1_ref[...], preferred_element_type=jnp.float32)
    y1_ref[...] = xw * dinv
    dinv_ref[...] = dinv


def _tca(x, W1, d0, d1):
    return pl.pallas_call(
        _tca_body,
        out_shape=(
            jax.ShapeDtypeStruct((N, DH), jnp.float32),
            jax.ShapeDtypeStruct((N, 1), jnp.float32),
        ),
    )(x, W1, d0, d1)


def _tcb_body(a0_ref, a1_ref, y1_ref, dinv_ref, b1_ref, w2_ref, y2_ref):
    dinv = dinv_ref[...]
    h1 = dinv * (a0_ref[...] + a1_ref[...] + y1_ref[...]) + b1_ref[...]
    h1 = jnp.maximum(h1, 0.0)
    y2_ref[...] = jnp.dot(h1, w2_ref[...],
                          preferred_element_type=jnp.float32) * dinv


def _tcb(a0, a1, y1, dinv, b1, W2):
    return pl.pallas_call(
        _tcb_body,
        out_shape=jax.ShapeDtypeStruct((N, DH), jnp.float32),
    )(a0, a1, y1, dinv, b1, W2)


def _tcc_body(a0_ref, a1_ref, y2_ref, dinv_ref, b2_ref, wl_ref, bl_ref,
              wh_ref, bh_ref, bt_ref, out_ref):
    h2 = dinv_ref[...] * (a0_ref[...] + a1_ref[...] + y2_ref[...]) + b2_ref[...]
    hl = jnp.dot(h2, wl_ref[...], preferred_element_type=jnp.float32) + bl_ref[...]
    gid = lax.broadcasted_iota(jnp.int32, (G, N), 0)
    mask = (gid == bt_ref[...]).astype(jnp.float32)
    pooled = jnp.dot(mask, hl, preferred_element_type=jnp.float32)
    cnt = jnp.sum(mask, axis=1, keepdims=True)
    pooled = pooled / jnp.maximum(cnt, 1.0)
    logits = jnp.dot(pooled, wh_ref[...],
                     preferred_element_type=jnp.float32) + bh_ref[...]
    m = jnp.max(logits, axis=1, keepdims=True)
    ex = jnp.exp(logits - m)
    out_ref[...] = ex / jnp.sum(ex, axis=1, keepdims=True)


def _tcc(a0, a1, y2, dinv, b2, Wl, bl, Wh, bh, bt):
    return pl.pallas_call(
        _tcc_body,
        out_shape=jax.ShapeDtypeStruct((G, DOUT), jnp.float32),
    )(a0, a1, y2, dinv, b2, Wl, bl, Wh, bh, bt)


def kernel(x, edge_index, edge_weight, batch_idx, W1, b1, W2, b2, Wl, bl,
           Wh, bh):
    src = edge_index[0]
    dst = edge_index[1]
    npad = EPAD - E
    pad = jnp.arange(npad, dtype=jnp.int32) % N
    srcp = jnp.concatenate([src, pad]).reshape(NW, 3, TH, C)
    dstp = jnp.concatenate([dst, pad]).reshape(NW, 3, TH, C)
    wp = jnp.concatenate(
        [edge_weight, jnp.zeros((npad,), jnp.float32)]).reshape(NW, 3, TH, C)

    d0, d1 = _deg_call(dstp, wp)
    y1, dinv = _tca(x, W1, d0.reshape(N, 1), d1.reshape(N, 1))
    a10, a11 = _agg_call(y1, srcp, dstp, wp)
    y2 = _tcb(a10, a11, y1, dinv, b1.reshape(1, DH), W2)
    a20, a21 = _agg_call(y2, srcp, dstp, wp)
    out = _tcc(a20, a21, y2, dinv, b2.reshape(1, DH), Wl, bl.reshape(1, DH // 2),
               Wh, bh.reshape(1, DOUT), batch_idx.reshape(1, N))
    return out
